# baseline (speedup 1.0000x reference)
import jax
import jax.numpy as jnp
from jax.experimental import pallas as pl
from jax.experimental.pallas import tpu as pltpu

_NH = 16
_NB = 32
_Q = 2048
_K = 2048
_LINE = 2 * _Q
_THR = (8, 12, 16, 23, 32, 46, 64, 91)
_NA = 16
_NR = 128


def _line_kernel(table_ref, line_ref):
    j = jax.lax.broadcasted_iota(jnp.int32, (1, _LINE), 1)
    d = j - (_Q - 1)
    a = jnp.abs(d)
    v = jnp.minimum(a, 7)
    for t in _THR:
        v = v + (a >= t).astype(jnp.int32)
    bucket = jnp.where(d > 0, 16, 0) + v
    acc = jnp.zeros((_NH, _LINE), jnp.float32)
    for b in range(_NB):
        col = table_ref[b, :].reshape(_NH, 1)
        acc = jnp.where(bucket == b, col, acc)
    line_ref[...] = acc


def _copies(u_ref, out_ref, sems, par, r):
    cps = []
    for a in range(_NA):
        cps.append(pltpu.make_async_copy(
            u_ref.at[par, :, pl.ds((_NA - 1 - a) * 128, _K)],
            out_ref.at[:, a * _NR + r, :],
            sems.at[par, a],
        ))
    return cps


_NBUF = 4


def _expand_kernel(line_ref, out_ref, u_ref, sems):
    c = pl.program_id(0)
    i = pl.program_id(1)
    r0 = c * (_NR // 2)
    par = jax.lax.rem(i, _NBUF)

    @pl.when(i >= _NBUF)
    def _():
        for cp in _copies(u_ref, out_ref, sems, par, r0 + i - _NBUF):
            cp.wait()

    shift = jax.lax.rem(jnp.int32(_LINE - 127) + r0 + i, jnp.int32(_LINE))
    u_ref[par] = pltpu.roll(line_ref[...], shift, 1)

    for cp in _copies(u_ref, out_ref, sems, par, r0 + i):
        cp.start()

    @pl.when(i == _NR // 2 - 1)
    def _():
        for back in range(_NBUF - 1, -1, -1):
            s = i - back
            p = jax.lax.rem(jnp.int32(s), _NBUF)
            for cp in _copies(u_ref, out_ref, sems, p, r0 + s):
                cp.wait()


def kernel(x, table):
    del x
    line = pl.pallas_call(
        _line_kernel,
        out_shape=jax.ShapeDtypeStruct((_NH, _LINE), jnp.float32),
    )(table)
    return pl.pallas_call(
        _expand_kernel,
        grid=(2, _NR // 2),
        in_specs=[pl.BlockSpec((_NH, _LINE), lambda c, r: (0, 0))],
        out_specs=pl.BlockSpec(memory_space=pl.ANY),
        out_shape=jax.ShapeDtypeStruct((_NH, _Q, _K), jnp.float32),
        scratch_shapes=[
            pltpu.VMEM((_NBUF, _NH, _LINE), jnp.float32),
            pltpu.SemaphoreType.DMA((_NBUF, _NA)),
        ],
        compiler_params=pltpu.CompilerParams(
            dimension_semantics=("parallel", "arbitrary"),
        ),
    )(line)
